# Initial kernel scaffold; baseline (speedup 1.0000x reference)
import jax
import jax.numpy as jnp
from jax.experimental import pallas as pl


def kernel(embed_user, embed_item, ui_vals, iu_vals, ui3_vals, user_js, ui_rows, ui_cols, iu_rows, iu_cols, ui3_rows, ui3_cols):
    raise NotImplementedError("write your pallas kernel here")


# speedup vs baseline: 3.6450x; 3.6450x over previous
import functools

import jax
import jax.numpy as jnp
from jax import lax
from jax.experimental import pallas as pl
from jax.experimental.pallas import tpu as pltpu
from jax.experimental.pallas import tpu_sc as plsc

U = 10000
I = 10000
D = 128
NNZ = 320000

NC = 2
NS = 16
NW = NC * NS

EB = 128
NBLK = NNZ // EB
BASE_BLK = NBLK // NW
REM = NBLK - BASE_BLK * NW
ROWS_PER_TILE = 624
_ZCHUNKS = (128, 128, 128, 128, 112)
_TAIL_BASE = ROWS_PER_TILE * NS
_TAIL_ROWS = U - _TAIL_BASE


def _scale_rows(gbuf, valsb):
    def group(g, carry):
        vv = valsb[pl.ds(g * 16, 16)]
        for l in range(16):
            v = vv[l]
            e = g * 16 + l
            for j in range(D // 16):
                gbuf[e, pl.ds(j * 16, 16)] = gbuf[e, pl.ds(j * 16, 16)] * v
        return carry

    lax.fori_loop(0, EB // 16, group, 0)


def _spmm_accumulate(rows_hbm, cols_hbm, vals_hbm, x_hbm, out_hbm,
                     acc, zbuf, colsb, ridxb, valsb, gbuf, sem,
                     c, s, start, count):
    rbase = s * ROWS_PER_TILE
    off = 0
    for nz in _ZCHUNKS:
        pltpu.sync_copy(zbuf.at[pl.ds(0, nz)], acc.at[pl.ds(rbase + off, nz)])
        off += nz

    @pl.when(s == NS - 1)
    def _zero_tail():
        pltpu.sync_copy(zbuf.at[pl.ds(0, _TAIL_ROWS)],
                        acc.at[pl.ds(_TAIL_BASE, _TAIL_ROWS)])

    plsc.subcore_barrier()

    def blk(b, carry):
        base = (start + b) * EB
        pltpu.sync_copy(cols_hbm.at[pl.ds(base, EB)], colsb)
        pltpu.sync_copy(rows_hbm.at[pl.ds(base, EB)], ridxb.at[0])
        pltpu.sync_copy(vals_hbm.at[pl.ds(base, EB)], valsb)
        pltpu.async_copy(x_hbm.at[colsb], gbuf, sem).wait()
        _scale_rows(gbuf, valsb)
        pltpu.sync_copy(gbuf, acc.at[ridxb.at[0]], add=True)
        return carry

    lax.fori_loop(0, count, blk, 0)
    plsc.subcore_barrier()

    pltpu.sync_copy(acc.at[pl.ds(rbase, ROWS_PER_TILE)],
                    out_hbm.at[c, pl.ds(rbase, ROWS_PER_TILE)])

    @pl.when(s == NS - 1)
    def _write_tail():
        pltpu.sync_copy(acc.at[pl.ds(_TAIL_BASE, _TAIL_ROWS)],
                        out_hbm.at[c, pl.ds(_TAIL_BASE, _TAIL_ROWS)])

    plsc.subcore_barrier()


def _tile_prologue(zbuf):
    c = lax.axis_index("c")
    s = lax.axis_index("s")
    wid = s * NC + c
    start = wid * BASE_BLK + jnp.minimum(wid, REM)
    count = BASE_BLK + (wid < REM).astype(jnp.int32)

    def zrow(r, carry):
        for j in range(D // 16):
            zbuf[r, pl.ds(j * 16, 16)] = jnp.zeros((16,), jnp.float32)
        return carry

    lax.fori_loop(0, 128, zrow, 0)
    return c, s, start, count


_SC_SCRATCH = [
    pltpu.VMEM_SHARED((U, D), jnp.float32),
    pltpu.VMEM((128, D), jnp.float32),
    pltpu.VMEM((EB,), jnp.int32),
    pltpu.VMEM((1, EB), jnp.int32),
    pltpu.VMEM((EB,), jnp.float32),
    pltpu.VMEM((EB, D), jnp.float32),
    pltpu.SemaphoreType.DMA,
]

_MESH = plsc.VectorSubcoreMesh(core_axis_name="c", subcore_axis_name="s")


@functools.partial(
    pl.kernel,
    out_type=[jax.ShapeDtypeStruct((NC, U, D), jnp.float32)] * 4,
    mesh=_MESH,
    scratch_types=_SC_SCRATCH,
)
def _phase_a(eu, ei, ui_r, ui_c, ui_v, iu_r, iu_c, iu_v, u3_r, u3_c, u3_v,
             p_g1u, p_g1i, p_g3u, p_g3i,
             acc, zbuf, colsb, ridxb, valsb, gbuf, sem):
    c, s, start, count = _tile_prologue(zbuf)
    args = (acc, zbuf, colsb, ridxb, valsb, gbuf, sem, c, s, start, count)
    _spmm_accumulate(ui_r, ui_c, ui_v, ei, p_g1u, *args)
    _spmm_accumulate(iu_r, iu_c, iu_v, eu, p_g1i, *args)
    _spmm_accumulate(u3_r, u3_c, u3_v, ei, p_g3u, *args)
    _spmm_accumulate(u3_c, u3_r, u3_v, eu, p_g3i, *args)


@functools.partial(
    pl.kernel,
    out_type=[jax.ShapeDtypeStruct((NC, U, D), jnp.float32)] * 2,
    mesh=_MESH,
    scratch_types=_SC_SCRATCH,
)
def _phase_b(g1u, g1i, ui_r, ui_c, ui_v, iu_r, iu_c, iu_v,
             p_g2u, p_g2i,
             acc, zbuf, colsb, ridxb, valsb, gbuf, sem):
    c, s, start, count = _tile_prologue(zbuf)
    args = (acc, zbuf, colsb, ridxb, valsb, gbuf, sem, c, s, start, count)
    _spmm_accumulate(ui_r, ui_c, ui_v, g1i, p_g2u, *args)
    _spmm_accumulate(iu_r, iu_c, iu_v, g1u, p_g2i, *args)


_RB = 1000
_GRID = U // _RB


def _combine1_body(p1u, p1i, g1u, g1i):
    g1u[...] = p1u[0] + p1u[1]
    g1i[...] = p1i[0] + p1i[1]


def _combine1(p_g1u, p_g1i):
    return pl.pallas_call(
        _combine1_body,
        grid=(_GRID,),
        in_specs=[pl.BlockSpec((NC, _RB, D), lambda i: (0, i, 0))] * 2,
        out_specs=[pl.BlockSpec((_RB, D), lambda i: (i, 0))] * 2,
        out_shape=[jax.ShapeDtypeStruct((U, D), jnp.float32)] * 2,
    )(p_g1u, p_g1i)


def _combine2_body(eu, ei, g1u, g1i, p2u, p2i, p3u, p3i, ujs, ou, oi):
    g3u = p3u[0] + p3u[1]
    ou[...] = 0.25 * (eu[...] + g1u[...] + (p2u[0] + p2u[1])) + g3u * ujs[...]
    oi[...] = 0.25 * (ei[...] + g1i[...] + (p2i[0] + p2i[1])
                      + (p3i[0] + p3i[1]))


def _combine2(eu, ei, g1u, g1i, p_g2u, p_g2i, p_g3u, p_g3i, user_js):
    dense = pl.BlockSpec((_RB, D), lambda i: (i, 0))
    part = pl.BlockSpec((NC, _RB, D), lambda i: (0, i, 0))
    return pl.pallas_call(
        _combine2_body,
        grid=(_GRID,),
        in_specs=[dense, dense, dense, dense, part, part, part, part,
                  pl.BlockSpec((_RB, 1), lambda i: (i, 0))],
        out_specs=[dense, dense],
        out_shape=[jax.ShapeDtypeStruct((U, D), jnp.float32)] * 2,
    )(eu, ei, g1u, g1i, p_g2u, p_g2i, p_g3u, p_g3i, user_js)


def kernel(embed_user, embed_item, ui_vals, iu_vals, ui3_vals, user_js,
           ui_rows, ui_cols, iu_rows, iu_cols, ui3_rows, ui3_cols):
    ui_r = ui_rows.astype(jnp.int32)
    ui_c = ui_cols.astype(jnp.int32)
    iu_r = iu_rows.astype(jnp.int32)
    iu_c = iu_cols.astype(jnp.int32)
    u3_r = ui3_rows.astype(jnp.int32)
    u3_c = ui3_cols.astype(jnp.int32)

    p_g1u, p_g1i, p_g3u, p_g3i = _phase_a(
        embed_user, embed_item,
        ui_r, ui_c, ui_vals, iu_r, iu_c, iu_vals, u3_r, u3_c, ui3_vals)
    g1u, g1i = _combine1(p_g1u, p_g1i)
    p_g2u, p_g2i = _phase_b(g1u, g1i, ui_r, ui_c, ui_vals, iu_r, iu_c, iu_vals)
    return _combine2(embed_user, embed_item, g1u, g1i,
                     p_g2u, p_g2i, p_g3u, p_g3i, user_js)
